# baseline (speedup 1.0000x reference)
import jax
import jax.numpy as jnp
from jax import lax
from jax.experimental import pallas as pl
from jax.experimental.pallas import tpu as pltpu
from jax.experimental.pallas import tpu_sc as plsc

VOCAB = 100000
EMBED = 64
B = 4096
S = 50
SP = 56
NC, NS = 2, 16
NW = NC * NS
BW = B // NW
K = 8
NB = BW // K


def _emb_kernel(idx_hbm, table_hbm, out_hbm, idx_v, g_a, g_b, sem_a, sem_b):
    wid = lax.axis_index("s") * NC + lax.axis_index("c")
    b0 = wid * BW

    pltpu.sync_copy(idx_hbm.at[wid], idx_v)

    def fire(batch, g, sem):
        for j in range(K):
            pltpu.async_copy(table_hbm.at[idx_v.at[batch * K + j]],
                             g.at[j], sem)

    def drain(g, sem):
        for j in range(K):
            pltpu.make_async_copy(table_hbm.at[idx_v.at[0]], g.at[j],
                                  sem).wait()

    def write(batch, g):
        for j in range(K):
            pltpu.sync_copy(g.at[j, pl.ds(0, S)],
                            out_hbm.at[b0 + batch * K + j])

    fire(0, g_a, sem_a)

    def pair(t, carry):
        fire(2 * t + 1, g_b, sem_b)
        drain(g_a, sem_a)
        write(2 * t, g_a)

        @pl.when(t < NB // 2 - 1)
        def _():
            fire(2 * t + 2, g_a, sem_a)

        drain(g_b, sem_b)
        write(2 * t + 1, g_b)
        return carry

    lax.fori_loop(0, NB // 2, pair, 0)


@jax.jit
def _emb(idx, table):
    f = pl.kernel(
        _emb_kernel,
        out_type=jax.ShapeDtypeStruct((B, S, EMBED), jnp.float32),
        mesh=plsc.VectorSubcoreMesh(core_axis_name="c", subcore_axis_name="s"),
        scratch_types=[
            pltpu.VMEM((BW, SP), jnp.int32),
            pltpu.VMEM((K, SP, EMBED), jnp.float32),
            pltpu.VMEM((K, SP, EMBED), jnp.float32),
            pltpu.SemaphoreType.DMA,
            pltpu.SemaphoreType.DMA,
        ],
        compiler_params=pltpu.CompilerParams(use_tc_tiling_on_sc=False),
    )
    return f(idx, table)


def kernel(indices, weight):
    idx = indices.astype(jnp.int32)
    idxp = jnp.pad(idx, ((0, 0), (0, SP - S))).reshape(NW, BW, SP)
    return _emb(idxp, weight)


# speedup vs baseline: 1.0018x; 1.0018x over previous
import jax
import jax.numpy as jnp
from jax import lax
from jax.experimental import pallas as pl
from jax.experimental.pallas import tpu as pltpu
from jax.experimental.pallas import tpu_sc as plsc

VOCAB = 100000
EMBED = 64
B = 4096
S = 50
SP = 56
NC, NS = 2, 16
NW = NC * NS
BW = B // NW
K = 8
NB = BW // K


def _emb_kernel(idx_hbm, table_hbm, out_hbm, idx_v, rows_a, rows_b, sem_a,
                sem_b, sem_w):
    wid = lax.axis_index("s") * NC + lax.axis_index("c")
    b0 = wid * BW

    pltpu.sync_copy(idx_hbm.at[wid], idx_v)

    def fire(batch, rows, sem):
        for j in range(K):
            pltpu.async_copy(table_hbm.at[idx_v.at[batch * K + j]],
                             rows.at[j], sem)

    def drain(rows, sem):
        for j in range(K):
            pltpu.make_async_copy(table_hbm.at[idx_v.at[0]], rows.at[j],
                                  sem).wait()

    def write(batch, rows):
        cs = [pltpu.async_copy(rows.at[j, pl.ds(0, S)],
                               out_hbm.at[b0 + batch * K + j], sem_w)
              for j in range(K)]
        for c in cs:
            c.wait()

    fire(0, rows_a, sem_a)

    def pair(t, carry):
        fire(2 * t + 1, rows_b, sem_b)
        drain(rows_a, sem_a)
        write(2 * t, rows_a)

        @pl.when(t < NB // 2 - 1)
        def _():
            fire(2 * t + 2, rows_a, sem_a)

        drain(rows_b, sem_b)
        write(2 * t + 1, rows_b)
        return carry

    lax.fori_loop(0, NB // 2, pair, 0)


@jax.jit
def _emb(idx, table):
    f = pl.kernel(
        _emb_kernel,
        out_type=jax.ShapeDtypeStruct((B, S, EMBED), jnp.float32),
        mesh=plsc.VectorSubcoreMesh(core_axis_name="c", subcore_axis_name="s"),
        scratch_types=[
            pltpu.VMEM((BW, SP), jnp.int32),
            pltpu.VMEM((K, SP, EMBED), jnp.float32),
            pltpu.VMEM((K, SP, EMBED), jnp.float32),
            pltpu.SemaphoreType.DMA,
            pltpu.SemaphoreType.DMA,
            pltpu.SemaphoreType.DMA,
        ],
        compiler_params=pltpu.CompilerParams(use_tc_tiling_on_sc=False),
    )
    return f(idx, table)


def kernel(indices, weight):
    idx = indices.astype(jnp.int32)
    idxp = jnp.pad(idx, ((0, 0), (0, SP - S))).reshape(NW, BW, SP)
    return _emb(idxp, weight)


# speedup vs baseline: 3.2113x; 3.2056x over previous
import functools

import jax
import jax.numpy as jnp
from jax import lax
from jax.experimental import pallas as pl
from jax.experimental.pallas import tpu as pltpu
from jax.experimental.pallas import tpu_sc as plsc

VOCAB = 100000
EMBED = 64
TOTAL = 4096 * 50
NC, NS = 2, 16
NW = NC * NS
PER_W = TOTAL // NW
GROUP = 128
NG = PER_W // GROUP
K = 5
NB = NG // K


def _emb_kernel(idx_hbm, table_hbm, out_hbm, idx_v, rows_a, rows_b, sem_a,
                sem_b):
    wid = lax.axis_index("s") * NC + lax.axis_index("c")
    pltpu.sync_copy(idx_hbm.at[wid], idx_v)

    def fire(batch, rows, sem):
        for j in range(K):
            pltpu.async_copy(table_hbm.at[idx_v.at[batch * K + j]],
                             rows.at[j], sem)

    def drain(rows, sem):
        for j in range(K):
            pltpu.make_async_copy(table_hbm.at[idx_v.at[0]], rows.at[j],
                                  sem).wait()

    fire(0, rows_a, sem_a)

    def pair(t, carry):
        fire(2 * t + 1, rows_b, sem_b)
        drain(rows_a, sem_a)
        pltpu.sync_copy(rows_a, out_hbm.at[wid, pl.ds(2 * t * K, K)])

        @pl.when(t < NB // 2 - 1)
        def _():
            fire(2 * t + 2, rows_a, sem_a)

        drain(rows_b, sem_b)
        pltpu.sync_copy(rows_b, out_hbm.at[wid, pl.ds((2 * t + 1) * K, K)])
        return carry

    lax.fori_loop(0, NB // 2, pair, 0)


@jax.jit
def _emb(idx, table):
    f = pl.kernel(
        _emb_kernel,
        out_type=jax.ShapeDtypeStruct((NW, NG, GROUP, EMBED), jnp.float32),
        mesh=plsc.VectorSubcoreMesh(core_axis_name="c", subcore_axis_name="s"),
        scratch_types=[
            pltpu.VMEM((NG, GROUP), jnp.int32),
            pltpu.VMEM((K, GROUP, EMBED), jnp.float32),
            pltpu.VMEM((K, GROUP, EMBED), jnp.float32),
            pltpu.SemaphoreType.DMA,
            pltpu.SemaphoreType.DMA,
        ],
        compiler_params=pltpu.CompilerParams(use_tc_tiling_on_sc=False),
    )
    return f(idx, table)


def kernel(indices, weight):
    B, S = indices.shape
    idx = indices.astype(jnp.int32).reshape(NW, NG, GROUP)
    out = _emb(idx, weight)
    return out.reshape(B, S, EMBED)
